# baseline (speedup 1.0000x reference)
import jax
import jax.numpy as jnp
from jax import lax
from jax.experimental import pallas as pl
from jax.experimental.pallas import tpu as pltpu
from jax.experimental.pallas import tpu_sc as plsc

NBITS = 256
HIDDEN = 256
SPATIAL = 32
B = 128
C = 128
HW = SPATIAL * SPATIAL

NC = 2
NS = 16
NW = NC * NS
BPW = B // NW
LANES = 16
TCH = 32
SCH = 64
SP_UNROLL = 32
NGRP = HIDDEN // (2 * TCH)


def _sc_body(msg_hbm, lat_hbm, w_hbm, out_hbm,
             msg_v, idx_v, rows_v, acc_v, bld_v,
             sem_g, sem_si, sem_so, sem_r0, sem_r1, sem_b0, sem_b1):
    cid = lax.axis_index("c")
    sid = lax.axis_index("s")
    wid = sid * NC + cid
    lane = lax.iota(jnp.int32, LANES)
    nh = HIDDEN // LANES
    half = NBITS // 2

    def _batch(j, _):
        b = wid * BPW + j

        lin_s = pltpu.make_async_copy(
            lat_hbm.at[b], out_hbm.at[b, pl.ds(0, C)], sem_si)
        lin_s.start()

        pltpu.sync_copy(msg_hbm.at[b], msg_v)
        for t in range(NBITS // LANES):
            idx_v[pl.ds(t * LANES, LANES)] = (
                2 * (t * LANES + lane) + msg_v[pl.ds(t * LANES, LANES)])

        def _gather(h):
            return pltpu.make_async_copy(
                w_hbm.at[idx_v.at[pl.ds(h * half, half)]], rows_v, sem_g)

        def _rows(lo):
            def _row(r, accs):
                return tuple(
                    accs[t] + rows_v[r, pl.ds(t * LANES, LANES)]
                    for t in range(nh))
            return _row

        _gather(0).start()
        _gather(0).wait()
        accs = tuple(jnp.zeros((LANES,), jnp.float32) for _ in range(nh))
        accs = lax.fori_loop(0, half, _rows(0), accs)
        _gather(1).start()
        _gather(1).wait()
        accs = lax.fori_loop(0, half, _rows(1), accs)
        for t in range(nh):
            acc_v[pl.ds(t * LANES, LANES)] = accs[t]

        n_sp = HW // LANES // SP_UNROLL

        def _bout(g, s):
            sem = sem_b0 if s == 0 else sem_b1
            return pltpu.make_async_copy(
                bld_v.at[s],
                out_hbm.at[b, pl.ds(C + g * 2 * TCH + s * TCH, TCH)], sem)

        def _build(g, s):
            for cl in range(TCH // LANES):
                base = g * 2 * TCH + s * TCH + cl * LANES
                vec = acc_v[pl.ds(base, LANES)]
                for k in range(LANES):
                    r = cl * LANES + k
                    splat = vec * 0.0 + vec[k]

                    def _sp(t, _, r=r, s=s, splat=splat):
                        for u in range(SP_UNROLL):
                            bld_v[s, r,
                                  pl.ds((t * SP_UNROLL + u) * LANES,
                                        LANES)] = splat
                        return 0

                    lax.fori_loop(0, n_sp, _sp, 0)

        def _grp(g, _):
            @pl.when(g > 0)
            def _():
                _bout(g - 1, 0).wait()
            _build(g, 0)
            _bout(g, 0).start()

            @pl.when(g > 0)
            def _():
                _bout(g - 1, 1).wait()
            _build(g, 1)
            _bout(g, 1).start()
            return 0

        lax.fori_loop(0, NGRP, _grp, 0)
        _bout(NGRP - 1, 0).wait()
        _bout(NGRP - 1, 1).wait()
        lin_s.wait()
        return 0

    lax.fori_loop(0, BPW, _batch, 0)


def kernel(latents, msg, W):
    lat3 = latents.reshape(B, C, HW)
    mesh = plsc.VectorSubcoreMesh(core_axis_name="c", subcore_axis_name="s")
    out = pl.kernel(
        _sc_body,
        out_type=jax.ShapeDtypeStruct((B, C + HIDDEN, HW), jnp.float32),
        mesh=mesh,
        scratch_types=[
            pltpu.VMEM((NBITS,), jnp.int32),
            pltpu.VMEM((NBITS,), jnp.int32),
            pltpu.VMEM((NBITS // 2, HIDDEN), jnp.float32),
            pltpu.VMEM((HIDDEN,), jnp.float32),
            pltpu.VMEM((2, TCH, HW), jnp.float32),
            pltpu.SemaphoreType.DMA,
            pltpu.SemaphoreType.DMA,
            pltpu.SemaphoreType.DMA,
            pltpu.SemaphoreType.DMA,
            pltpu.SemaphoreType.DMA,
            pltpu.SemaphoreType.DMA,
            pltpu.SemaphoreType.DMA,
        ],
    )(msg.astype(jnp.int32), lat3, W)
    return out.reshape(B, C + HIDDEN, SPATIAL, SPATIAL)


# speedup vs baseline: 5.1380x; 5.1380x over previous
import jax
import jax.numpy as jnp
from jax import lax
from jax.experimental import pallas as pl
from jax.experimental.pallas import tpu as pltpu
from jax.experimental.pallas import tpu_sc as plsc

NBITS = 256
HIDDEN = 256
SPATIAL = 32
B = 128
C = 128
HW = SPATIAL * SPATIAL

NC = 2
NS = 16
NW = NC * NS
BPW = B // NW
LANES = 16
TCH = 32
SCH = 64
SP_UNROLL = 32
NGRP = HIDDEN // (2 * TCH)


def _sc_body(msg_hbm, lat_hbm, w_hbm, out_hbm,
             msg_v, idx_v, rows_v, acc_v, bld_v,
             sem_g, sem_si, sem_so, sem_r0, sem_r1, sem_b0, sem_b1):
    cid = lax.axis_index("c")
    sid = lax.axis_index("s")
    wid = sid * NC + cid
    lane = lax.iota(jnp.int32, LANES)
    nh = HIDDEN // LANES
    half = NBITS // 2

    def _batch(j, _):
        b = wid * BPW + j

        def _rel_in(k, s):
            sem = sem_r0 if s == 0 else sem_r1
            return pltpu.make_async_copy(
                lat_hbm.at[b, pl.ds(k * TCH, TCH)], bld_v.at[s], sem)

        def _rel_out(k, s):
            sem = sem_r0 if s == 0 else sem_r1
            return pltpu.make_async_copy(
                bld_v.at[s], out_hbm.at[b, pl.ds(k * TCH, TCH)], sem)

        _rel_in(0, 0).start()
        _rel_in(1, 1).start()

        pltpu.sync_copy(msg_hbm.at[b], msg_v)
        for t in range(NBITS // LANES):
            idx_v[pl.ds(t * LANES, LANES)] = (
                2 * (t * LANES + lane) + msg_v[pl.ds(t * LANES, LANES)])

        def _gather(h):
            return pltpu.make_async_copy(
                w_hbm.at[idx_v.at[pl.ds(h * half, half)]], rows_v, sem_g)

        def _rows(lo):
            def _row(r, accs):
                return tuple(
                    accs[t] + rows_v[r, pl.ds(t * LANES, LANES)]
                    for t in range(nh))
            return _row

        _gather(0).start()
        _gather(0).wait()
        accs = tuple(jnp.zeros((LANES,), jnp.float32) for _ in range(nh))
        accs = lax.fori_loop(0, half, _rows(0), accs)
        _gather(1).start()

        _rel_in(0, 0).wait()
        _rel_out(0, 0).start()
        _rel_in(1, 1).wait()
        _rel_out(1, 1).start()

        _gather(1).wait()
        accs = lax.fori_loop(0, half, _rows(1), accs)
        for t in range(nh):
            acc_v[pl.ds(t * LANES, LANES)] = accs[t]

        _rel_out(0, 0).wait()
        _rel_in(2, 0).start()
        _rel_out(1, 1).wait()
        _rel_in(3, 1).start()
        _rel_in(2, 0).wait()
        _rel_out(2, 0).start()
        _rel_in(3, 1).wait()
        _rel_out(3, 1).start()
        _rel_out(2, 0).wait()
        _rel_out(3, 1).wait()

        n_sp = HW // LANES // SP_UNROLL

        def _bout(g, s):
            sem = sem_b0 if s == 0 else sem_b1
            return pltpu.make_async_copy(
                bld_v.at[s],
                out_hbm.at[b, pl.ds(C + g * 2 * TCH + s * TCH, TCH)], sem)

        def _build(g, s):
            for cl in range(TCH // LANES):
                base = g * 2 * TCH + s * TCH + cl * LANES
                vec = acc_v[pl.ds(base, LANES)]
                for k in range(LANES):
                    r = cl * LANES + k
                    splat = vec * 0.0 + vec[k]

                    def _sp(t, _, r=r, s=s, splat=splat):
                        for u in range(SP_UNROLL):
                            bld_v[s, r,
                                  pl.ds((t * SP_UNROLL + u) * LANES,
                                        LANES)] = splat
                        return 0

                    lax.fori_loop(0, n_sp, _sp, 0)

        def _grp(g, _):
            @pl.when(g > 0)
            def _():
                _bout(g - 1, 0).wait()
            _build(g, 0)
            _bout(g, 0).start()

            @pl.when(g > 0)
            def _():
                _bout(g - 1, 1).wait()
            _build(g, 1)
            _bout(g, 1).start()
            return 0

        lax.fori_loop(0, NGRP, _grp, 0)
        _bout(NGRP - 1, 0).wait()
        _bout(NGRP - 1, 1).wait()
        return 0

    lax.fori_loop(0, BPW, _batch, 0)


def kernel(latents, msg, W):
    lat3 = latents.reshape(B, C, HW)
    mesh = plsc.VectorSubcoreMesh(core_axis_name="c", subcore_axis_name="s")
    out = pl.kernel(
        _sc_body,
        out_type=jax.ShapeDtypeStruct((B, C + HIDDEN, HW), jnp.float32),
        mesh=mesh,
        scratch_types=[
            pltpu.VMEM((NBITS,), jnp.int32),
            pltpu.VMEM((NBITS,), jnp.int32),
            pltpu.VMEM((NBITS // 2, HIDDEN), jnp.float32),
            pltpu.VMEM((HIDDEN,), jnp.float32),
            pltpu.VMEM((2, TCH, HW), jnp.float32),
            pltpu.SemaphoreType.DMA,
            pltpu.SemaphoreType.DMA,
            pltpu.SemaphoreType.DMA,
            pltpu.SemaphoreType.DMA,
            pltpu.SemaphoreType.DMA,
            pltpu.SemaphoreType.DMA,
            pltpu.SemaphoreType.DMA,
        ],
    )(msg.astype(jnp.int32), lat3, W)
    return out.reshape(B, C + HIDDEN, SPATIAL, SPATIAL)


# speedup vs baseline: 5.1543x; 1.0032x over previous
import jax
import jax.numpy as jnp
from jax import lax
from jax.experimental import pallas as pl
from jax.experimental.pallas import tpu as pltpu
from jax.experimental.pallas import tpu_sc as plsc

NBITS = 256
HIDDEN = 256
SPATIAL = 32
B = 128
C = 128
HW = SPATIAL * SPATIAL

NC = 2
NS = 16
NW = NC * NS
BPW = B // NW
LANES = 16
TCH = 32
SCH = 64
SP_UNROLL = 32
NGRP = HIDDEN // (2 * TCH)


def _sc_body(msg_hbm, lat_hbm, w_hbm, out_hbm,
             msg_v, idx_v, rows_v, acc_v, bld_v,
             sem_g, sem_si, sem_so, sem_r0, sem_r1, sem_b0, sem_b1):
    cid = lax.axis_index("c")
    sid = lax.axis_index("s")
    wid = sid * NC + cid
    lane = lax.iota(jnp.int32, LANES)
    nh = HIDDEN // LANES
    half = NBITS // 2

    def _batch(j, _):
        b = wid * BPW + j

        def _rel_in(k, s):
            sem = sem_r0 if s == 0 else sem_r1
            return pltpu.make_async_copy(
                lat_hbm.at[b, pl.ds(k * TCH, TCH)], bld_v.at[s], sem)

        def _rel_out(k, s):
            sem = sem_r0 if s == 0 else sem_r1
            return pltpu.make_async_copy(
                bld_v.at[s], out_hbm.at[b, pl.ds(k * TCH, TCH)], sem)

        _rel_in(0, 0).start()
        _rel_in(1, 1).start()

        pltpu.sync_copy(msg_hbm.at[b], msg_v)
        for t in range(NBITS // LANES):
            idx_v[pl.ds(t * LANES, LANES)] = (
                2 * (t * LANES + lane) + msg_v[pl.ds(t * LANES, LANES)])

        def _gather(h):
            return pltpu.make_async_copy(
                w_hbm.at[idx_v.at[pl.ds(h * half, half)]], rows_v, sem_g)

        def _rows(lo):
            def _row(r, accs):
                return tuple(
                    accs[t] + rows_v[r, pl.ds(t * LANES, LANES)]
                    for t in range(nh))
            return _row

        _gather(0).start()
        _gather(0).wait()
        accs = tuple(jnp.zeros((LANES,), jnp.float32) for _ in range(nh))
        accs = lax.fori_loop(0, half, _rows(0), accs)
        _gather(1).start()

        _rel_in(0, 0).wait()
        _rel_out(0, 0).start()
        _rel_in(1, 1).wait()
        _rel_out(1, 1).start()

        _gather(1).wait()
        accs = lax.fori_loop(0, half, _rows(1), accs)
        for t in range(nh):
            acc_v[pl.ds(t * LANES, LANES)] = accs[t]

        _rel_out(0, 0).wait()
        _rel_in(2, 0).start()
        _rel_out(1, 1).wait()
        _rel_in(3, 1).start()
        _rel_in(2, 0).wait()
        _rel_out(2, 0).start()
        _rel_in(3, 1).wait()
        _rel_out(3, 1).start()
        _rel_out(2, 0).wait()
        _rel_out(3, 1).wait()

        n_sp = HW // LANES // SP_UNROLL

        def _bout(g, s):
            sem = sem_b0 if s == 0 else sem_b1
            return pltpu.make_async_copy(
                bld_v.at[s],
                out_hbm.at[b, pl.ds(C + g * 2 * TCH + s * TCH, TCH)], sem)

        def _build(g, s):
            for cl in range(TCH // LANES):
                base = g * 2 * TCH + s * TCH + cl * LANES
                vec = acc_v[pl.ds(base, LANES)]
                for k in range(LANES):
                    r = cl * LANES + k
                    splat = vec * 0.0 + vec[k]

                    def _sp(t, _, r=r, s=s, splat=splat):
                        for u in range(SP_UNROLL):
                            bld_v[s, r,
                                  pl.ds((t * SP_UNROLL + u) * LANES,
                                        LANES)] = splat
                        return 0

                    lax.fori_loop(0, n_sp, _sp, 0)

        def _grp(g, _):
            @pl.when(g > 0)
            def _():
                _bout(g - 1, 0).wait()
            _build(g, 0)
            _bout(g, 0).start()

            @pl.when(g > 0)
            def _():
                _bout(g - 1, 1).wait()
            _build(g, 1)
            _bout(g, 1).start()
            return 0

        lax.fori_loop(0, NGRP, _grp, 0)
        _bout(NGRP - 1, 0).wait()
        _bout(NGRP - 1, 1).wait()
        return 0

    lax.fori_loop(0, BPW, _batch, 0)


def kernel(latents, msg, W):
    lat3 = latents.reshape(B, C, HW)
    mesh = plsc.VectorSubcoreMesh(core_axis_name="c", subcore_axis_name="s", num_cores=2)
    out = pl.kernel(
        _sc_body,
        out_type=jax.ShapeDtypeStruct((B, C + HIDDEN, HW), jnp.float32),
        mesh=mesh,
        scratch_types=[
            pltpu.VMEM((NBITS,), jnp.int32),
            pltpu.VMEM((NBITS,), jnp.int32),
            pltpu.VMEM((NBITS // 2, HIDDEN), jnp.float32),
            pltpu.VMEM((HIDDEN,), jnp.float32),
            pltpu.VMEM((2, TCH, HW), jnp.float32),
            pltpu.SemaphoreType.DMA,
            pltpu.SemaphoreType.DMA,
            pltpu.SemaphoreType.DMA,
            pltpu.SemaphoreType.DMA,
            pltpu.SemaphoreType.DMA,
            pltpu.SemaphoreType.DMA,
            pltpu.SemaphoreType.DMA,
        ],
    )(msg.astype(jnp.int32), lat3, W)
    return out.reshape(B, C + HIDDEN, SPATIAL, SPATIAL)
